# baseline (speedup 1.0000x reference)
import functools

import jax
import jax.numpy as jnp
from jax import lax
from jax.experimental import pallas as pl
from jax.experimental.pallas import tpu as pltpu
from jax.experimental.pallas import tpu_sc as plsc

N_TOKENS = 32768
N_EXPERTS = 64
TOPK = 8
LANES = 16

NUM_CORES = 2
NUM_SUBCORES = 16
NW = NUM_CORES * NUM_SUBCORES
ROWS_PER_W = N_TOKENS // NW

EBLK = N_EXPERTS // 8
TBLK = N_TOKENS // 128
TBLK_PER_W = TBLK // NW
IN_WORDS_PER_W = ROWS_PER_W * N_EXPERTS
OUT_WORDS_PER_W = ROWS_PER_W * TOPK
EBLK_STRIDE_HBM = TBLK * 1024
EBLK_STRIDE_V = TBLK_PER_W * 1024

_mesh = plsc.VectorSubcoreMesh(
    core_axis_name="c", subcore_axis_name="s",
    num_cores=NUM_CORES, num_subcores=NUM_SUBCORES)


def _merge_desc(a, ia, b, ib):
  rb = lax.rev(b, (0,))
  rib = lax.rev(ib, (0,))
  ge = a >= rb
  key = jnp.where(ge, a, rb)
  val = jnp.where(ge, ia, rib)
  return plsc.sort_key_val(key, val, descending=True)


@functools.partial(
    pl.kernel,
    out_type=[
        jax.ShapeDtypeStruct((N_TOKENS * TOPK,), jnp.int32),
        jax.ShapeDtypeStruct((N_TOKENS * TOPK,), jnp.float32),
    ],
    mesh=_mesh,
    scratch_types=[
        pltpu.VMEM((IN_WORDS_PER_W,), jnp.float32),
        pltpu.VMEM((OUT_WORDS_PER_W,), jnp.int32),
        pltpu.VMEM((OUT_WORDS_PER_W,), jnp.float32),
    ],
    compiler_params=pltpu.CompilerParams(needs_layout_passes=False),
)
def _route(logits_hbm, out_idx_hbm, out_val_hbm, logits_v, idx_v, val_v):
  wid = lax.axis_index("s") * NUM_CORES + lax.axis_index("c")
  tb0 = wid * TBLK_PER_W
  for b in range(EBLK):
    pltpu.sync_copy(
        logits_hbm.at[pl.ds(b * EBLK_STRIDE_HBM + tb0 * 1024, EBLK_STRIDE_V)],
        logits_v.at[pl.ds(b * EBLK_STRIDE_V, EBLK_STRIDE_V)])

  iota = lax.iota(jnp.int32, LANES)
  mask8 = iota < TOPK
  gbase = []
  for k in range(4):
    e = iota + k * LANES
    gbase.append((e >> 3) * EBLK_STRIDE_V + (e & 7) * 128)
  sbase = iota * 128

  @plsc.parallel_loop(0, ROWS_PER_W, 1, unroll=4)
  def body(t):
    toff = (t >> 7) * 896 + t
    g0 = gbase[0] + toff
    g1 = gbase[1] + toff
    g2 = gbase[2] + toff
    g3 = gbase[3] + toff
    v0 = plsc.load_gather(logits_v, [g0])
    v1 = plsc.load_gather(logits_v, [g1])
    v2 = plsc.load_gather(logits_v, [g2])
    v3 = plsc.load_gather(logits_v, [g3])

    s0, i0 = plsc.sort_key_val(v0, iota, descending=True)
    s1, i1 = plsc.sort_key_val(v1, iota + LANES, descending=True)
    s2, i2 = plsc.sort_key_val(v2, iota + 2 * LANES, descending=True)
    s3, i3 = plsc.sort_key_val(v3, iota + 3 * LANES, descending=True)
    m01k, m01i = _merge_desc(s0, i0, s1, i1)
    m23k, m23i = _merge_desc(s2, i2, s3, i3)
    mk, mi = _merge_desc(m01k, m01i, m23k, m23i)

    denom = jnp.sum(jnp.exp(v0) + jnp.exp(v1) + jnp.exp(v2) + jnp.exp(v3))
    probs = jnp.exp(mk) / denom

    so = sbase + toff
    plsc.store_scatter(idx_v, [so], mi, mask=mask8)
    plsc.store_scatter(val_v, [so], probs, mask=mask8)

  out_off = wid * OUT_WORDS_PER_W
  pltpu.sync_copy(idx_v, out_idx_hbm.at[pl.ds(out_off, OUT_WORDS_PER_W)])
  pltpu.sync_copy(val_v, out_val_hbm.at[pl.ds(out_off, OUT_WORDS_PER_W)])


def kernel(router_logits):
  x_flat = (router_logits.T
            .reshape(EBLK, 8, TBLK, 128)
            .transpose(0, 2, 1, 3)
            .reshape(-1))
  idx_flat, val_flat = _route(x_flat)
  idx = idx_flat.reshape(TBLK, TOPK, 128).transpose(0, 2, 1).reshape(
      N_TOKENS, TOPK)
  val = val_flat.reshape(TBLK, TOPK, 128).transpose(0, 2, 1).reshape(
      N_TOKENS, TOPK)
  return (idx, val)


# speedup vs baseline: 1.0710x; 1.0710x over previous
import functools

import jax
import jax.numpy as jnp
from jax import lax
from jax.experimental import pallas as pl
from jax.experimental.pallas import tpu as pltpu
from jax.experimental.pallas import tpu_sc as plsc

N_TOKENS = 32768
N_EXPERTS = 64
TOPK = 8
LANES = 16

NUM_CORES = 2
NUM_SUBCORES = 16
NW = NUM_CORES * NUM_SUBCORES
ROWS_PER_W = N_TOKENS // NW

EBLK = N_EXPERTS // 8
TBLK = N_TOKENS // 128
TBLK_PER_W = TBLK // NW
IN_WORDS_PER_W = ROWS_PER_W * N_EXPERTS
OUT_WORDS_PER_W = ROWS_PER_W * TOPK
EBLK_STRIDE_HBM = TBLK * 1024
EBLK_STRIDE_V = TBLK_PER_W * 1024

_mesh = plsc.VectorSubcoreMesh(
    core_axis_name="c", subcore_axis_name="s",
    num_cores=NUM_CORES, num_subcores=NUM_SUBCORES)


def _merge_desc(a, ia, b, ib):
  rb = lax.rev(b, (0,))
  rib = lax.rev(ib, (0,))
  ge = a >= rb
  key = jnp.where(ge, a, rb)
  val = jnp.where(ge, ia, rib)
  return plsc.sort_key_val(key, val, descending=True)


@functools.partial(
    pl.kernel,
    out_type=[
        jax.ShapeDtypeStruct((N_TOKENS * TOPK,), jnp.int32),
        jax.ShapeDtypeStruct((N_TOKENS * TOPK,), jnp.float32),
    ],
    mesh=_mesh,
    scratch_types=[
        pltpu.VMEM((IN_WORDS_PER_W,), jnp.float32),
        pltpu.VMEM((OUT_WORDS_PER_W,), jnp.int32),
        pltpu.VMEM((OUT_WORDS_PER_W,), jnp.float32),
        pltpu.SemaphoreType.DMA,
    ],
    compiler_params=pltpu.CompilerParams(needs_layout_passes=False),
)
def _route(logits_hbm, out_idx_hbm, out_val_hbm, logits_v, idx_v, val_v, sem):
  wid = lax.axis_index("s") * NUM_CORES + lax.axis_index("c")
  tb0 = wid * TBLK_PER_W
  copies = [
      pltpu.async_copy(
          logits_hbm.at[pl.ds(b * EBLK_STRIDE_HBM + tb0 * 1024,
                              EBLK_STRIDE_V)],
          logits_v.at[pl.ds(b * EBLK_STRIDE_V, EBLK_STRIDE_V)],
          sem)
      for b in range(EBLK)
  ]
  for c in copies:
    c.wait()

  iota = lax.iota(jnp.int32, LANES)
  mask8 = iota < TOPK
  gbase = []
  for k in range(4):
    e = iota + k * LANES
    gbase.append((e >> 3) * EBLK_STRIDE_V + (e & 7) * 128)
  sbase = iota * 128

  @plsc.parallel_loop(0, ROWS_PER_W, 1, unroll=4)
  def body(t):
    toff = (t >> 7) * 896 + t
    g0 = gbase[0] + toff
    g1 = gbase[1] + toff
    g2 = gbase[2] + toff
    g3 = gbase[3] + toff
    v0 = plsc.load_gather(logits_v, [g0])
    v1 = plsc.load_gather(logits_v, [g1])
    v2 = plsc.load_gather(logits_v, [g2])
    v3 = plsc.load_gather(logits_v, [g3])

    s0, i0 = plsc.sort_key_val(v0, iota, descending=True)
    s1, i1 = plsc.sort_key_val(v1, iota + LANES, descending=True)
    s2, i2 = plsc.sort_key_val(v2, iota + 2 * LANES, descending=True)
    s3, i3 = plsc.sort_key_val(v3, iota + 3 * LANES, descending=True)
    m01k, m01i = _merge_desc(s0, i0, s1, i1)
    m23k, m23i = _merge_desc(s2, i2, s3, i3)
    mk, mi = _merge_desc(m01k, m01i, m23k, m23i)

    denom = jnp.sum(jnp.exp(v0) + jnp.exp(v1) + jnp.exp(v2) + jnp.exp(v3))
    probs = jnp.exp(mk) / denom

    so = sbase + toff
    plsc.store_scatter(idx_v, [so], mi, mask=mask8)
    plsc.store_scatter(val_v, [so], probs, mask=mask8)

  out_off = wid * OUT_WORDS_PER_W
  pltpu.sync_copy(idx_v, out_idx_hbm.at[pl.ds(out_off, OUT_WORDS_PER_W)])
  pltpu.sync_copy(val_v, out_val_hbm.at[pl.ds(out_off, OUT_WORDS_PER_W)])


def kernel(router_logits):
  x_flat = (router_logits.T
            .reshape(EBLK, 8, TBLK, 128)
            .transpose(0, 2, 1, 3)
            .reshape(-1))
  idx_flat, val_flat = _route(x_flat)
  idx = idx_flat.reshape(TBLK, TOPK, 128).transpose(0, 2, 1).reshape(
      N_TOKENS, TOPK)
  val = val_flat.reshape(TBLK, TOPK, 128).transpose(0, 2, 1).reshape(
      N_TOKENS, TOPK)
  return (idx, val)
